# baseline (speedup 1.0000x reference)
import functools

import jax
import jax.numpy as jnp
from jax import lax
from jax.experimental import pallas as pl
from jax.experimental.pallas import tpu as pltpu
from jax.experimental.pallas import tpu_sc as plsc

_BATCH = 16384
_DIM = 32
_PACK = 4
_NC = 2
_NS = 16
_NW = _NC * _NS
_BPW = _BATCH // _NW
_L = 16
_HALF = _BPW // 2
_GPH = _HALF // _L


def _body(uid, iid, ut, it, out, uidx, iidx, uq, iq, urows, irows, outv,
          semu, semi):
    wid = lax.axis_index("s") * _NC + lax.axis_index("c")
    base = wid * _BPW
    pltpu.sync_copy(uid.at[pl.ds(base, _BPW)], uidx)
    pltpu.sync_copy(iid.at[pl.ds(base, _BPW)], iidx)

    rows0 = lax.iota(jnp.int32, _L)

    for h in range(2):
        def shift_ids(i, carry):
            s = pl.ds(h * _HALF + i * _L, _L)
            uq[pl.ds(i * _L, _L)] = lax.shift_right_logical(uidx[s], 2)
            iq[pl.ds(i * _L, _L)] = lax.shift_right_logical(iidx[s], 2)
            return carry

        lax.fori_loop(0, _GPH, shift_ids, 0)

        cu = pltpu.async_copy(ut.at[uq], urows, semu)
        ci = pltpu.async_copy(it.at[iq], irows, semi)
        cu.wait()
        ci.wait()

        def compute_group(g, carry):
            rows = rows0 + g * _L
            s = pl.ds(h * _HALF + g * _L, _L)
            uoff = lax.shift_left(jnp.bitwise_and(uidx[s], 3), 5)
            ioff = lax.shift_left(jnp.bitwise_and(iidx[s], 3), 5)
            acc = jnp.zeros((_L,), jnp.float32)
            for d in range(_DIM):
                uv = plsc.load_gather(urows, [rows, uoff + d])
                iv = plsc.load_gather(irows, [rows, ioff + d])
                acc = acc + uv * iv
            outv[s] = acc
            return carry

        lax.fori_loop(0, _GPH, compute_group, 0)

    pltpu.sync_copy(outv, out.at[pl.ds(base, _BPW)])


_course_rec = functools.partial(
    pl.kernel,
    out_type=jax.ShapeDtypeStruct((_BATCH,), jnp.float32),
    mesh=plsc.VectorSubcoreMesh(core_axis_name="c", subcore_axis_name="s"),
    compiler_params=pltpu.CompilerParams(needs_layout_passes=False),
    scratch_types=[
        pltpu.VMEM((_BPW,), jnp.int32),
        pltpu.VMEM((_BPW,), jnp.int32),
        pltpu.VMEM((_HALF,), jnp.int32),
        pltpu.VMEM((_HALF,), jnp.int32),
        pltpu.VMEM((_HALF, _PACK * _DIM), jnp.float32),
        pltpu.VMEM((_HALF, _PACK * _DIM), jnp.float32),
        pltpu.VMEM((_BPW,), jnp.float32),
        pltpu.SemaphoreType.DMA,
        pltpu.SemaphoreType.DMA,
    ],
)(_body)


def kernel(user_ids, item_ids, user_table, item_table):
    ut = jnp.reshape(user_table, (-1, _PACK * _DIM))
    it = jnp.reshape(item_table, (-1, _PACK * _DIM))
    return _course_rec(user_ids, item_ids, ut, it)


# speedup vs baseline: 1.5852x; 1.5852x over previous
import functools

import jax
import jax.numpy as jnp
from jax import lax
from jax.experimental import pallas as pl
from jax.experimental.pallas import tpu as pltpu
from jax.experimental.pallas import tpu_sc as plsc

_BATCH = 16384
_DIM = 32
_NC = 2
_NS = 16
_NW = _NC * _NS
_BPW = _BATCH // _NW
_L = 16
_G = _BPW // _L
_ROWS = _BPW // 128


def _body(uid, iid, ut, it, out, uidx, iidx, urows, irows,
          outv, semu0, semi0, semu1, semi1):
    wid = lax.axis_index("s") * _NC + lax.axis_index("c")
    wrow = wid * _ROWS
    for r in range(_ROWS):
        pltpu.sync_copy(uid.at[wrow + r], uidx.at[pl.ds(r * 128, 128)])
        pltpu.sync_copy(iid.at[wrow + r], iidx.at[pl.ds(r * 128, 128)])

    lanes = lax.iota(jnp.int32, _L)
    sems = ((semu0, semi0), (semu1, semi1))

    def fetch_group(g, par):
        su, si = sems[par]
        uvec = uidx[pl.ds(g * _L, _L)]
        ivec = iidx[pl.ds(g * _L, _L)]
        for j in range(_L):
            slot = par * _L + j
            pltpu.async_copy(ut.at[pl.ds(uvec[j], 1)], urows.at[pl.ds(slot, 1)], su)
            pltpu.async_copy(it.at[pl.ds(ivec[j], 1)], irows.at[pl.ds(slot, 1)], si)

    def drain_group(par):
        su, si = sems[par]
        for j in range(_L):
            slot = par * _L + j
            pltpu.make_async_copy(ut.at[pl.ds(0, 1)], urows.at[pl.ds(slot, 1)], su).wait()
            pltpu.make_async_copy(it.at[pl.ds(0, 1)], irows.at[pl.ds(slot, 1)], si).wait()

    def compute_group(g, par):
        acc = jnp.zeros((_L,), jnp.float32)
        for j in range(_L):
            slot = par * _L + j
            u0 = urows[slot, pl.ds(0, _L)]
            u1 = urows[slot, pl.ds(_L, _L)]
            v0 = irows[slot, pl.ds(0, _L)]
            v1 = irows[slot, pl.ds(_L, _L)]
            s = u0 * v0 + u1 * v1
            acc = jnp.where(lanes == j, jnp.sum(s), acc)
        outv[pl.ds(g * _L, _L)] = acc

    def step(k, carry):
        g0 = 2 * k
        fetch_group(g0, 0)
        fetch_group(g0 + 1, 1)
        drain_group(0)
        compute_group(g0, 0)
        drain_group(1)
        compute_group(g0 + 1, 1)
        return carry

    lax.fori_loop(0, _G // 2, step, 0)

    for r in range(_ROWS):
        pltpu.sync_copy(outv.at[pl.ds(r * 128, 128)], out.at[wrow + r])


_course_rec = functools.partial(
    pl.kernel,
    out_type=jax.ShapeDtypeStruct((128, 128), jnp.float32),
    mesh=plsc.VectorSubcoreMesh(core_axis_name="c", subcore_axis_name="s"),
    compiler_params=pltpu.CompilerParams(needs_layout_passes=False),
    scratch_types=[
        pltpu.VMEM((_BPW,), jnp.int32),
        pltpu.VMEM((_BPW,), jnp.int32),
        pltpu.VMEM((2 * _L, _DIM), jnp.float32),
        pltpu.VMEM((2 * _L, _DIM), jnp.float32),
        pltpu.VMEM((_BPW,), jnp.float32),
        pltpu.SemaphoreType.DMA,
        pltpu.SemaphoreType.DMA,
        pltpu.SemaphoreType.DMA,
        pltpu.SemaphoreType.DMA,
    ],
)(_body)


def kernel(user_ids, item_ids, user_table, item_table):
    uid = jnp.reshape(user_ids, (128, 128))
    iid = jnp.reshape(item_ids, (128, 128))
    out = _course_rec(uid, iid, user_table, item_table)
    return jnp.reshape(out, (_BATCH,))


# speedup vs baseline: 1.5865x; 1.0008x over previous
import functools

import jax
import jax.numpy as jnp
from jax import lax
from jax.experimental import pallas as pl
from jax.experimental.pallas import tpu as pltpu
from jax.experimental.pallas import tpu_sc as plsc

_BATCH = 16384
_DIM = 32
_NC = 2
_NS = 16
_NW = _NC * _NS
_BPW = _BATCH // _NW
_L = 16
_G = _BPW // _L
_ROWS = _BPW // 128


def _body(uid, iid, ut, it, out, uidx, iidx, urows, irows,
          outv, semu0, semi0, semu1, semi1):
    wid = lax.axis_index("s") * _NC + lax.axis_index("c")
    wrow = wid * _ROWS
    for r in range(_ROWS):
        pltpu.sync_copy(uid.at[wrow + r], uidx.at[pl.ds(r * 128, 128)])
        pltpu.sync_copy(iid.at[wrow + r], iidx.at[pl.ds(r * 128, 128)])

    lanes = lax.iota(jnp.int32, _L)
    sems = ((semu0, semi0), (semu1, semi1))

    def fetch_group(g, par):
        su, si = sems[par]
        uvec = uidx[pl.ds(g * _L, _L)]
        ivec = iidx[pl.ds(g * _L, _L)]
        for j in range(_L):
            slot = par * _L + j
            pltpu.async_copy(ut.at[pl.ds(uvec[j], 1)], urows.at[pl.ds(slot, 1)], su)
            pltpu.async_copy(it.at[pl.ds(ivec[j], 1)], irows.at[pl.ds(slot, 1)], si)

    def drain_group(par):
        su, si = sems[par]
        for j in range(_L):
            slot = par * _L + j
            pltpu.make_async_copy(ut.at[pl.ds(0, 1)], urows.at[pl.ds(slot, 1)], su).wait()
            pltpu.make_async_copy(it.at[pl.ds(0, 1)], irows.at[pl.ds(slot, 1)], si).wait()

    def compute_group(g, par):
        acc = jnp.zeros((_L,), jnp.float32)
        for j in range(_L):
            slot = par * _L + j
            u0 = urows[slot, pl.ds(0, _L)]
            u1 = urows[slot, pl.ds(_L, _L)]
            v0 = irows[slot, pl.ds(0, _L)]
            v1 = irows[slot, pl.ds(_L, _L)]
            s = u0 * v0 + u1 * v1
            acc = jnp.where(lanes == j, jnp.sum(s), acc)
        outv[pl.ds(g * _L, _L)] = acc

    def step(k, carry):
        g0 = 2 * k
        fetch_group(g0, 0)
        fetch_group(g0 + 1, 1)
        drain_group(0)
        compute_group(g0, 0)
        drain_group(1)
        compute_group(g0 + 1, 1)
        return carry

    lax.fori_loop(0, _G // 2, step, 0)

    for r in range(_ROWS):
        pltpu.sync_copy(outv.at[pl.ds(r * 128, 128)], out.at[wrow + r])


_course_rec = functools.partial(
    pl.kernel,
    out_type=jax.ShapeDtypeStruct((128, 128), jnp.float32),
    mesh=plsc.VectorSubcoreMesh(core_axis_name="c", subcore_axis_name="s"),
    compiler_params=pltpu.CompilerParams(
        needs_layout_passes=False, use_tc_tiling_on_sc=True
    ),
    scratch_types=[
        pltpu.VMEM((_BPW,), jnp.int32),
        pltpu.VMEM((_BPW,), jnp.int32),
        pltpu.VMEM((2 * _L, _DIM), jnp.float32),
        pltpu.VMEM((2 * _L, _DIM), jnp.float32),
        pltpu.VMEM((_BPW,), jnp.float32),
        pltpu.SemaphoreType.DMA,
        pltpu.SemaphoreType.DMA,
        pltpu.SemaphoreType.DMA,
        pltpu.SemaphoreType.DMA,
    ],
)(_body)


def kernel(user_ids, item_ids, user_table, item_table):
    uid = jnp.reshape(user_ids, (128, 128))
    iid = jnp.reshape(item_ids, (128, 128))
    out = _course_rec(uid, iid, user_table, item_table)
    return jnp.reshape(out, (_BATCH,))
